# baseline (speedup 1.0000x reference)
import functools
import math

import jax
import jax.numpy as jnp
from jax import lax
from jax.experimental import pallas as pl
from jax.experimental.pallas import tpu as pltpu
from jax.experimental.pallas import tpu_sc as plsc

N_NODES = 1000000
N_DIM = 16
R = 10.0
BATCH = 16384

_NC = 2
_NS = 16
_NW = _NC * _NS
_TOTAL = 2 * BATCH
_PER_W = _TOTAL // _NW
_CHUNK = 128
_NCHUNK = _PER_W // _CHUNK


def _sc_gather(table, idx2d):
    mesh = plsc.VectorSubcoreMesh(core_axis_name="c", subcore_axis_name="s")

    @functools.partial(
        pl.kernel,
        out_type=jax.ShapeDtypeStruct((_TOTAL, N_DIM), jnp.float32),
        mesh=mesh,
        compiler_params=pltpu.CompilerParams(use_tc_tiling_on_sc=False),
        scratch_types=[
            pltpu.VMEM((_NCHUNK, _CHUNK), jnp.int32),
            pltpu.VMEM((_PER_W, N_DIM), jnp.float32),
            pltpu.SemaphoreType.DMA,
        ],
    )
    def k(table_hbm, idx_hbm, out_hbm, idx_v, rows_v, sem):
        wid = lax.axis_index("s") * _NC + lax.axis_index("c")
        pltpu.sync_copy(idx_hbm.at[pl.ds(wid * _NCHUNK, _NCHUNK)], idx_v)
        copies = [
            pltpu.async_copy(
                table_hbm.at[idx_v.at[j]],
                rows_v.at[pl.ds(j * _CHUNK, _CHUNK)],
                sem,
            )
            for j in range(_NCHUNK)
        ]
        for c in copies:
            c.wait()
        pltpu.sync_copy(rows_v, out_hbm.at[pl.ds(wid * _PER_W, _PER_W)])

    return k(table, idx2d)


def _tc_loss(rows, labels, beta):
    blk = 2048
    nblk = BATCH // blk
    const = N_DIM * math.log(2.0 * math.pi)
    inv = 1.0 / (N_NODES - 1)

    def body(beta_ref, u_ref, v_ref, y_ref, o_ref):
        u = u_ref[...]
        v = v_ref[...]
        du = u - v
        d2 = jnp.sum(du * du, axis=1) + 1e-12
        dist = jnp.sqrt(d2)
        z = beta_ref[0] * (dist - R)
        y = y_ref[...].astype(jnp.float32)
        loss = y * jnp.logaddexp(0.0, z) + (1.0 - y) * jnp.logaddexp(0.0, -z)
        t = jnp.sum(u * u, axis=1) + jnp.sum(v * v, axis=1)
        o_ref[...] = loss + (const + 0.5 * t) * inv

    return pl.pallas_call(
        body,
        grid=(nblk,),
        in_specs=[
            pl.BlockSpec(memory_space=pltpu.SMEM),
            pl.BlockSpec((blk, N_DIM), lambda i: (i, 0)),
            pl.BlockSpec((blk, N_DIM), lambda i: (i + nblk, 0)),
            pl.BlockSpec((blk,), lambda i: (i,)),
        ],
        out_specs=pl.BlockSpec((blk,), lambda i: (i,)),
        out_shape=jax.ShapeDtypeStruct((BATCH,), jnp.float32),
    )(jnp.reshape(beta, (1,)).astype(jnp.float32), rows, rows, labels)


def kernel(pairs, labels, table, beta):
    idx_flat = pairs.T.reshape(-1)
    idx2d = idx_flat.reshape(_TOTAL // _CHUNK, _CHUNK)
    rows = _sc_gather(table, idx2d)
    return _tc_loss(rows, labels, beta)


# speedup vs baseline: 4.1316x; 4.1316x over previous
import functools
import math

import jax
import jax.numpy as jnp
from jax import lax
from jax.experimental import pallas as pl
from jax.experimental.pallas import tpu as pltpu
from jax.experimental.pallas import tpu_sc as plsc

N_NODES = 1000000
N_DIM = 16
R = 10.0
BATCH = 16384

_NC = 2
_NS = 16
_NW = _NC * _NS
_E = 2 * BATCH
_EPW = _E // _NW
_G = _EPW // 16
_WIN = 1024
_TAIL = 640
_TB = N_NODES - _TAIL
_NWIN = _TB // _WIN + 1


def _sc_gather(table_t, tail_t, sidx, spos):
    mesh = plsc.VectorSubcoreMesh(core_axis_name="c", subcore_axis_name="s")

    @functools.partial(
        pl.kernel,
        out_type=jax.ShapeDtypeStruct((_E * N_DIM,), jnp.float32),
        mesh=mesh,
        compiler_params=pltpu.CompilerParams(needs_layout_passes=False),
        scratch_types=[
            pltpu.VMEM((_EPW,), jnp.int32),
            pltpu.VMEM((_EPW,), jnp.int32),
            pltpu.VMEM((N_DIM, _WIN), jnp.float32),
            pltpu.VMEM((N_DIM, _TAIL), jnp.float32),
            pltpu.VMEM((16, N_DIM), jnp.float32),
            pltpu.SemaphoreType.DMA,
        ],
    )
    def k(tab_hbm, tail_hbm, sidx_hbm, spos_hbm, out_hbm,
          idx_v, pos_v, win_v, tail_v, stage_v, sem):
        wid = lax.axis_index("s") * _NC + lax.axis_index("c")
        base = wid * _EPW
        pltpu.sync_copy(sidx_hbm.at[pl.ds(base, _EPW)], idx_v)
        pltpu.sync_copy(spos_hbm.at[pl.ds(base, _EPW)], pos_v)
        pltpu.sync_copy(tail_hbm, tail_v)

        lanes = lax.iota(jnp.int32, 16)

        def drain16(i, _):
            pltpu.make_async_copy(
                stage_v.at[0], out_hbm.at[pl.ds(0, N_DIM)], sem
            ).wait()
            return ()

        def group(g, cur):
            iu = idx_v[pl.ds(g * 16, 16)]
            ip = pos_v[pl.ds(g * 16, 16)]
            lax.cond(g > 0,
                     lambda: lax.fori_loop(0, 16, drain16, ()),
                     lambda: ())
            for l in range(16):
                r = iu[l]
                p = ip[l]
                tail = r >= _TB
                wneed = lax.select(tail, cur, r // _WIN)

                @pl.when(wneed != cur)
                def _():
                    wstart = pl.multiple_of(wneed * _WIN, 128)
                    pltpu.sync_copy(
                        tab_hbm.at[:, pl.ds(wstart, _WIN)], win_v)

                cur = wneed
                c_win = jnp.full((16,), lax.rem(r, _WIN), jnp.int32)
                c_tail = jnp.full(
                    (16,), lax.max(r - _TB, 0), jnp.int32)
                col_w = plsc.load_gather(win_v, [lanes, c_win])
                col_t = plsc.load_gather(tail_v, [lanes, c_tail])
                stage_v[l, :] = jnp.where(tail, col_t, col_w)
                pltpu.async_copy(
                    stage_v.at[l], out_hbm.at[pl.ds(p * N_DIM, N_DIM)], sem)
            return cur

        lax.fori_loop(0, _G, group, jnp.int32(-1))
        lax.fori_loop(0, 16, drain16, ())

    return k(table_t, tail_t, sidx, spos)


def _tc_loss(rows1d, labels2d, beta):
    const = N_DIM * math.log(2.0 * math.pi)
    inv = 1.0 / (N_NODES - 1)
    blk = 2048
    nblk = BATCH // blk
    rows = blk * N_DIM // 128

    def body(beta_ref, u_ref, v_ref, y_ref, o_ref):
        u = u_ref[...].reshape(rows, 128)
        v = v_ref[...].reshape(rows, 128)
        bd = (lax.broadcasted_iota(jnp.int32, (128, 8), 0) // N_DIM
              == lax.broadcasted_iota(jnp.int32, (128, 8), 1)
              ).astype(jnp.float32)
        du = u - v
        d2 = jnp.dot(du * du, bd, preferred_element_type=jnp.float32)
        t = jnp.dot(u * u + v * v, bd, preferred_element_type=jnp.float32)
        dist = jnp.sqrt(d2 + 1e-12)
        z = beta_ref[0] * (dist - R)
        y = y_ref[...].astype(jnp.float32)
        loss = y * jnp.logaddexp(0.0, z) + (1.0 - y) * jnp.logaddexp(0.0, -z)
        o_ref[...] = loss + (const + 0.5 * t) * inv

    return pl.pallas_call(
        body,
        grid=(nblk,),
        in_specs=[
            pl.BlockSpec(memory_space=pltpu.SMEM),
            pl.BlockSpec((blk * N_DIM,), lambda i: (i,)),
            pl.BlockSpec((blk * N_DIM,), lambda i: (i + nblk,)),
            pl.BlockSpec((rows, 8), lambda i: (i, 0)),
        ],
        out_specs=pl.BlockSpec((rows, 8), lambda i: (i, 0)),
        out_shape=jax.ShapeDtypeStruct((BATCH // 8, 8), jnp.float32),
    )(jnp.reshape(beta, (1,)).astype(jnp.float32), rows1d, rows1d, labels2d)


def kernel(pairs, labels, table, beta):
    table_t = table.T
    tail_t = table_t[:, _TB:]
    idx_flat = pairs.T.reshape(-1)
    pos = lax.iota(jnp.int32, _E)
    sidx, spos = lax.sort_key_val(idx_flat, pos)
    rows1d = _sc_gather(table_t, tail_t, sidx, spos)
    loss2d = _tc_loss(rows1d, labels.reshape(BATCH // 8, 8), beta)
    return loss2d.reshape(BATCH)


# speedup vs baseline: 4.6912x; 1.1354x over previous
import functools
import math

import jax
import jax.numpy as jnp
from jax import lax
from jax.experimental import pallas as pl
from jax.experimental.pallas import tpu as pltpu
from jax.experimental.pallas import tpu_sc as plsc

N_NODES = 1000000
N_DIM = 16
R = 10.0
BATCH = 16384

_NC = 2
_NS = 16
_NW = _NC * _NS
_E = 2 * BATCH
_EPW = _E // _NW
_G = _EPW // 16
_WIN = 2048
_TAIL = 640
_TB = N_NODES - _TAIL
_WMAX = (N_NODES - _WIN) // _WIN


def _sc_gather(table_t, tail_t, sidx, spos):
    mesh = plsc.VectorSubcoreMesh(core_axis_name="c", subcore_axis_name="s")

    @functools.partial(
        pl.kernel,
        out_type=jax.ShapeDtypeStruct((_E * N_DIM,), jnp.float32),
        mesh=mesh,
        compiler_params=pltpu.CompilerParams(needs_layout_passes=False),
        scratch_types=[
            pltpu.VMEM((_EPW,), jnp.int32),
            pltpu.VMEM((_EPW,), jnp.int32),
            pltpu.VMEM((N_DIM, _WIN), jnp.float32),
            pltpu.VMEM((N_DIM, _WIN), jnp.float32),
            pltpu.VMEM((N_DIM, _TAIL), jnp.float32),
            pltpu.VMEM((_EPW * N_DIM,), jnp.float32),
            pltpu.SemaphoreType.DMA,
            pltpu.SemaphoreType.DMA,
            pltpu.SemaphoreType.DMA,
        ],
    )
    def k(tab_hbm, tail_hbm, sidx_hbm, spos_hbm, out_hbm,
          idx_v, pos_v, win_a, win_b, tail_v, cols_v, sem_a, sem_b, sem_o):
        wid = lax.axis_index("s") * _NC + lax.axis_index("c")
        base = wid * _EPW
        pltpu.sync_copy(sidx_hbm.at[pl.ds(base, _EPW)], idx_v)
        pltpu.sync_copy(spos_hbm.at[pl.ds(base, _EPW)], pos_v)
        pltpu.sync_copy(tail_hbm, tail_v)

        lanes = lax.iota(jnp.int32, 16)

        def fetch_sync(w):
            ws = pl.multiple_of(w * _WIN, 128)

            @pl.when(lax.rem(w, 2) == 0)
            def _():
                pltpu.sync_copy(tab_hbm.at[:, pl.ds(ws, _WIN)], win_a)

            @pl.when(lax.rem(w, 2) == 1)
            def _():
                pltpu.sync_copy(tab_hbm.at[:, pl.ds(ws, _WIN)], win_b)

        def fetch_async(w):
            ws = pl.multiple_of(w * _WIN, 128)

            @pl.when(lax.rem(w, 2) == 0)
            def _():
                pltpu.async_copy(
                    tab_hbm.at[:, pl.ds(ws, _WIN)], win_a, sem_a)

            @pl.when(lax.rem(w, 2) == 1)
            def _():
                pltpu.async_copy(
                    tab_hbm.at[:, pl.ds(ws, _WIN)], win_b, sem_b)

        def wait_win(w):
            @pl.when(lax.rem(w, 2) == 0)
            def _():
                pltpu.make_async_copy(
                    tab_hbm.at[:, pl.ds(0, _WIN)], win_a, sem_a).wait()

            @pl.when(lax.rem(w, 2) == 1)
            def _():
                pltpu.make_async_copy(
                    tab_hbm.at[:, pl.ds(0, _WIN)], win_b, sem_b).wait()

        w0 = jnp.minimum(idx_v[pl.ds(0, 16)][0] // _WIN, _WMAX)
        fetch_sync(w0)
        pf0 = jnp.minimum(w0 + 1, _WMAX)
        fetch_async(pf0)

        def group(g, carry):
            cur, pf = carry
            iu = idx_v[pl.ds(g * 16, 16)]
            ip = pos_v[pl.ds(g * 16, 16)]
            for l in range(16):
                r = iu[l]
                p = ip[l]
                tail = r >= _TB
                wneed = lax.select(tail, cur, r // _WIN)
                trans = wneed != cur

                @pl.when(trans)
                def _():
                    wait_win(pf)

                @pl.when(trans & (wneed != pf))
                def _():
                    fetch_sync(wneed)

                pfid = jnp.minimum(wneed + 1, _WMAX)

                @pl.when(trans)
                def _():
                    fetch_async(pfid)

                cur = lax.select(trans, wneed, cur)
                pf = lax.select(trans, pfid, pf)

                c_win = jnp.full((16,), lax.rem(r, _WIN), jnp.int32)
                c_tail = jnp.full((16,), lax.max(r - _TB, 0), jnp.int32)
                col_a = plsc.load_gather(win_a, [lanes, c_win])
                col_b = plsc.load_gather(win_b, [lanes, c_win])
                col_t = plsc.load_gather(tail_v, [lanes, c_tail])
                col = jnp.where(
                    tail, col_t,
                    jnp.where(lax.rem(cur, 2) == 0, col_a, col_b))
                j = g * 16 + l
                cols_v[pl.ds(j * N_DIM, N_DIM)] = col
                pltpu.async_copy(
                    cols_v.at[pl.ds(j * N_DIM, N_DIM)],
                    out_hbm.at[pl.ds(p * N_DIM, N_DIM)], sem_o)
            return (cur, pf)

        _, pf_end = lax.fori_loop(0, _G, group, (w0, pf0))
        wait_win(pf_end)

        def drain_out(i, _):
            pltpu.make_async_copy(
                cols_v.at[pl.ds(0, N_DIM)],
                out_hbm.at[pl.ds(0, N_DIM)], sem_o).wait()
            return ()

        lax.fori_loop(0, _EPW, drain_out, ())

    return k(table_t, tail_t, sidx, spos)


def _tc_loss(rows1d, labels2d, beta):
    const = N_DIM * math.log(2.0 * math.pi)
    inv = 1.0 / (N_NODES - 1)
    blk = 2048
    nblk = BATCH // blk
    rows = blk * N_DIM // 128

    def body(beta_ref, u_ref, v_ref, y_ref, o_ref):
        u = u_ref[...].reshape(rows, 128)
        v = v_ref[...].reshape(rows, 128)
        bd = (lax.broadcasted_iota(jnp.int32, (128, 8), 0) // N_DIM
              == lax.broadcasted_iota(jnp.int32, (128, 8), 1)
              ).astype(jnp.float32)
        du = u - v
        d2 = jnp.dot(du * du, bd, preferred_element_type=jnp.float32)
        t = jnp.dot(u * u + v * v, bd, preferred_element_type=jnp.float32)
        dist = jnp.sqrt(d2 + 1e-12)
        z = beta_ref[0] * (dist - R)
        y = y_ref[...].astype(jnp.float32)
        loss = y * jnp.logaddexp(0.0, z) + (1.0 - y) * jnp.logaddexp(0.0, -z)
        o_ref[...] = loss + (const + 0.5 * t) * inv

    return pl.pallas_call(
        body,
        grid=(nblk,),
        in_specs=[
            pl.BlockSpec(memory_space=pltpu.SMEM),
            pl.BlockSpec((blk * N_DIM,), lambda i: (i,)),
            pl.BlockSpec((blk * N_DIM,), lambda i: (i + nblk,)),
            pl.BlockSpec((rows, 8), lambda i: (i, 0)),
        ],
        out_specs=pl.BlockSpec((rows, 8), lambda i: (i, 0)),
        out_shape=jax.ShapeDtypeStruct((BATCH // 8, 8), jnp.float32),
    )(jnp.reshape(beta, (1,)).astype(jnp.float32), rows1d, rows1d, labels2d)


def kernel(pairs, labels, table, beta):
    table_t = table.T
    tail_t = table_t[:, _TB:]
    idx_flat = pairs.T.reshape(-1)
    pos = lax.iota(jnp.int32, _E)
    sidx, spos = lax.sort_key_val(idx_flat, pos)
    rows1d = _sc_gather(table_t, tail_t, sidx, spos)
    loss2d = _tc_loss(rows1d, labels.reshape(BATCH // 8, 8), beta)
    return loss2d.reshape(BATCH)
